# baseline (speedup 1.0000x reference)
import functools

import jax
import jax.numpy as jnp
from jax import lax
from jax.experimental import pallas as pl
from jax.experimental.pallas import tpu as pltpu
from jax.experimental.pallas import tpu_sc as plsc

_NC = 2
_NS = 16
_NW = _NC * _NS
_B = 512
_KG = _B // 128
_L = 16


def _make_emb(hist, batch, dim):
  n_units = hist * (batch // _B)
  per_w = n_units // _NW
  chunks = batch // _B
  mesh = plsc.VectorSubcoreMesh(core_axis_name="c", subcore_axis_name="s")

  @functools.partial(
      pl.kernel,
      mesh=mesh,
      out_type=jax.ShapeDtypeStruct((hist, dim, batch), jnp.float32),
      scratch_types=[
          pltpu.VMEM((2, _B), jnp.int32),
          pltpu.VMEM((2, _B, dim), jnp.float32),
          pltpu.VMEM((2, dim, _B), jnp.float32),
          pltpu.SemaphoreType.DMA,
          pltpu.SemaphoreType.DMA,
          pltpu.SemaphoreType.DMA,
          pltpu.SemaphoreType.DMA,
          pltpu.SemaphoreType.DMA,
          pltpu.SemaphoreType.DMA,
      ],
      compiler_params=pltpu.CompilerParams(
          use_tc_tiling_on_sc=False, needs_layout_passes=False),
  )
  def emb(idx_hbm, table_hbm, out_hbm, idx_v, rows_v, rowst_v, isem0, isem1,
          gsem0, gsem1, ssem0, ssem1):
    wid = lax.axis_index("s") * _NC + lax.axis_index("c")
    u0 = wid * per_w
    isems = (isem0, isem1)
    gsems = (gsem0, gsem1)
    ssems = (ssem0, ssem1)
    lane_iota = lax.broadcasted_iota(jnp.int32, (_L,), 0)

    def unit_hb(u):
      h = u // chunks
      b0 = (u % chunks) * _B
      return h, b0

    def start_idx(u, p):
      h, b0 = unit_hb(u)
      pltpu.async_copy(idx_hbm.at[h, pl.ds(b0, _B)], idx_v.at[p], isems[p])

    def wait_idx(p):
      pltpu.make_async_copy(
          idx_hbm.at[0, pl.ds(0, _B)], idx_v.at[p], isems[p]).wait()

    def fire_gathers(p):
      return [
          pltpu.async_copy(
              table_hbm.at[idx_v.at[p].at[pl.ds(k * 128, 128)]],
              rows_v.at[p].at[pl.ds(k * 128, 128)],
              gsems[p])
          for k in range(_KG)
      ]

    def transpose(p):
      src = rows_v.at[p]
      dst = rowst_v.at[p]

      @plsc.parallel_loop(0, _B // _L, 1, unroll=4)
      def tbody(g):
        row_idx = lane_iota + g * _L
        for e in range(dim):
          vals = plsc.load_gather(
              src, [row_idx, jnp.full((_L,), e, jnp.int32)])
          dst[e, pl.ds(g * _L, _L)] = vals

    def fire_stores(u, p):
      h, b0 = unit_hb(u)
      pltpu.async_copy(
          rowst_v.at[p], out_hbm.at[h].at[:, pl.ds(b0, _B)], ssems[p])

    def wait_stores(p):
      pltpu.make_async_copy(
          rowst_v.at[p], out_hbm.at[0].at[:, pl.ds(0, _B)], ssems[p]).wait()

    def drain_gathers(p):
      for k in range(_KG):
        pltpu.make_async_copy(
            table_hbm.at[idx_v.at[p].at[pl.ds(k * 128, 128)]],
            rows_v.at[p].at[pl.ds(k * 128, 128)],
            gsems[p]).wait()

    n_pairs = per_w // 2

    start_idx(u0, 0)
    start_idx(u0 + 1, 1)
    wait_idx(0)
    fire_gathers(0)

    def body(t, carry):
      ua = u0 + 2 * t
      wait_idx(1)
      fire_gathers(1)
      drain_gathers(0)

      @pl.when(t + 1 < n_pairs)
      def _():
        start_idx(ua + 2, 0)

      @pl.when(t > 0)
      def _():
        wait_stores(0)

      transpose(0)
      fire_stores(ua, 0)
      drain_gathers(1)

      @pl.when(t + 1 < n_pairs)
      def _():
        start_idx(ua + 3, 1)

      @pl.when(t > 0)
      def _():
        wait_stores(1)

      transpose(1)
      fire_stores(ua + 1, 1)

      @pl.when(t + 1 < n_pairs)
      def _():
        wait_idx(0)
        fire_gathers(0)

      return carry

    lax.fori_loop(0, n_pairs, body, 0)
    wait_stores(0)
    wait_stores(1)

  return emb


def kernel(label_ids, table):
  batch, hist = label_ids.shape
  vocab, dim = table.shape
  idx_t = label_ids.T.astype(jnp.int32)
  out_phys = _make_emb(hist, batch, dim)(idx_t, table)
  return jnp.transpose(out_phys, (2, 0, 1))


# speedup vs baseline: 1.0591x; 1.0591x over previous
import functools

import jax
import jax.numpy as jnp
from jax import lax
from jax.experimental import pallas as pl
from jax.experimental.pallas import tpu as pltpu
from jax.experimental.pallas import tpu_sc as plsc

_NC = 2
_NS = 16
_NW = _NC * _NS
_B = 512
_KG = _B // 128
_L = 16


def _make_emb(hist, batch, dim):
  n_units = hist * (batch // _B)
  per_w = n_units // _NW
  chunks = batch // _B
  mesh = plsc.VectorSubcoreMesh(core_axis_name="c", subcore_axis_name="s")

  @functools.partial(
      pl.kernel,
      mesh=mesh,
      out_type=jax.ShapeDtypeStruct((hist, dim, batch), jnp.float32),
      scratch_types=[
          pltpu.VMEM((2, _B), jnp.int32),
          pltpu.VMEM((2, _B, dim), jnp.float32),
          pltpu.VMEM((2, dim, _B), jnp.float32),
          pltpu.SemaphoreType.DMA,
          pltpu.SemaphoreType.DMA,
          pltpu.SemaphoreType.DMA,
          pltpu.SemaphoreType.DMA,
          pltpu.SemaphoreType.DMA,
          pltpu.SemaphoreType.DMA,
      ],
      compiler_params=pltpu.CompilerParams(
          use_tc_tiling_on_sc=False, needs_layout_passes=False),
  )
  def emb(idx_hbm, table_hbm, out_hbm, idx_v, rows_v, rowst_v, isem0, isem1,
          gsem0, gsem1, ssem0, ssem1):
    wid = lax.axis_index("s") * _NC + lax.axis_index("c")
    u0 = wid * per_w
    isems = (isem0, isem1)
    gsems = (gsem0, gsem1)
    ssems = (ssem0, ssem1)
    lane_iota = lax.broadcasted_iota(jnp.int32, (_L,), 0)

    def unit_hb(u):
      h = u // chunks
      b0 = (u % chunks) * _B
      return h, b0

    def start_idx(u, p):
      h, b0 = unit_hb(u)
      pltpu.async_copy(idx_hbm.at[h, pl.ds(b0, _B)], idx_v.at[p], isems[p])

    def wait_idx(p):
      pltpu.make_async_copy(
          idx_hbm.at[0, pl.ds(0, _B)], idx_v.at[p], isems[p]).wait()

    def fire_gathers(p):
      return [
          pltpu.async_copy(
              table_hbm.at[idx_v.at[p].at[pl.ds(k * 128, 128)]],
              rows_v.at[p].at[pl.ds(k * 128, 128)],
              gsems[p])
          for k in range(_KG)
      ]

    def transpose(p):
      src = rows_v.at[p]
      dst = rowst_v.at[p]

      @plsc.parallel_loop(0, _B // _L, 1, unroll=4)
      def tbody(g):
        row_idx = lane_iota + g * _L
        for e in range(dim):
          vals = plsc.load_gather(
              src, [row_idx, jnp.full((_L,), e, jnp.int32)])
          dst[e, pl.ds(g * _L, _L)] = vals

    def fire_stores(u, p):
      h, b0 = unit_hb(u)
      pltpu.async_copy(
          rowst_v.at[p], out_hbm.at[h].at[:, pl.ds(b0, _B)], ssems[p])

    def wait_stores(p):
      pltpu.make_async_copy(
          rowst_v.at[p], out_hbm.at[0].at[:, pl.ds(0, _B)], ssems[p]).wait()

    def drain_gathers(p):
      for k in range(_KG):
        pltpu.make_async_copy(
            table_hbm.at[idx_v.at[p].at[pl.ds(k * 128, 128)]],
            rows_v.at[p].at[pl.ds(k * 128, 128)],
            gsems[p]).wait()

    start_idx(u0, 0)
    start_idx(u0 + 1, 1)
    wait_idx(0)
    fire_gathers(0)
    drain_gathers(0)

    def body(j, carry):
      p = lax.rem(j, 2)

      @pl.when(j + 1 < per_w)
      def _():
        @pl.when(p == 0)
        def _():
          wait_idx(1)
          fire_gathers(1)

        @pl.when(p == 1)
        def _():
          wait_idx(0)
          fire_gathers(0)

      @pl.when(j + 2 < per_w)
      def _():
        @pl.when(p == 0)
        def _():
          start_idx(u0 + j + 2, 0)

        @pl.when(p == 1)
        def _():
          start_idx(u0 + j + 2, 1)

      @pl.when(j >= 2)
      def _():
        @pl.when(p == 0)
        def _():
          wait_stores(0)

        @pl.when(p == 1)
        def _():
          wait_stores(1)

      @pl.when(p == 0)
      def _():
        transpose(0)
        fire_stores(u0 + j, 0)

      @pl.when(p == 1)
      def _():
        transpose(1)
        fire_stores(u0 + j, 1)

      @pl.when(j + 1 < per_w)
      def _():
        @pl.when(p == 0)
        def _():
          drain_gathers(1)

        @pl.when(p == 1)
        def _():
          drain_gathers(0)

      return carry

    lax.fori_loop(0, per_w, body, 0)
    wait_stores(0)
    wait_stores(1)

  return emb


def kernel(label_ids, table):
  batch, hist = label_ids.shape
  vocab, dim = table.shape
  idx_t = label_ids.T.astype(jnp.int32)
  out_phys = _make_emb(hist, batch, dim)(idx_t, table)
  return jnp.transpose(out_phys, (2, 0, 1))
